# Initial kernel scaffold; baseline (speedup 1.0000x reference)
import jax
import jax.numpy as jnp
from jax.experimental import pallas as pl


def kernel(z, embedding):
    raise NotImplementedError("write your pallas kernel here")


# speedup vs baseline: 1.0079x; 1.0079x over previous
import functools

import jax
import jax.numpy as jnp
from jax import lax
from jax.experimental import pallas as pl
from jax.experimental.pallas import tpu as pltpu
from jax.experimental.pallas import tpu_sc as plsc

K = 8192
E = 32
T = 4096
BT = 256
NT = T // BT
CC = 0.25


def _vq_body(z_ref, e_ref, idx_ref, loss_ref, ppl_ref, counts_ref):
    i = pl.program_id(0)
    z = z_ref[...]
    e = e_ref[...]
    mm = lax.dot_general(z, e, (((1,), (1,)), ((), ())),
                         preferred_element_type=jnp.float32)
    z2 = jnp.sum(z * z, axis=1, keepdims=True)
    e2 = jnp.sum(e * e, axis=1)
    scores = z2 + e2[None, :] - 2.0 * mm
    minv = jnp.min(scores, axis=1)
    kiota = lax.broadcasted_iota(jnp.int32, (BT, K), 1)
    idx = jnp.min(jnp.where(scores == minv[:, None], kiota, K), axis=1)
    idx_ref[pl.ds(i * BT, BT)] = idx
    cnt = jnp.sum((idx[:, None] == kiota).astype(jnp.float32), axis=0)

    @pl.when(i == 0)
    def _():
        counts_ref[...] = cnt
        loss_ref[...] = jnp.sum(minv)[None, None]

    @pl.when(i > 0)
    def _():
        counts_ref[...] = counts_ref[...] + cnt
        loss_ref[...] = loss_ref[...] + jnp.sum(minv)[None, None]

    @pl.when(i == NT - 1)
    def _():
        p = counts_ref[...] * (1.0 / T)
        ppl_ref[...] = jnp.exp(-jnp.sum(p * jnp.log(p + 1e-10)))[None, None]
        loss_ref[...] = loss_ref[...] * (CC / (T * E))


_vq_call = pl.pallas_call(
    _vq_body,
    grid=(NT,),
    in_specs=[
        pl.BlockSpec((BT, E), lambda i: (i, 0)),
        pl.BlockSpec((K, E), lambda i: (0, 0)),
    ],
    out_specs=[
        pl.BlockSpec((T,), lambda i: (0,)),
        pl.BlockSpec((1, 1), lambda i: (0, 0)),
        pl.BlockSpec((1, 1), lambda i: (0, 0)),
    ],
    out_shape=[
        jax.ShapeDtypeStruct((T,), jnp.int32),
        jax.ShapeDtypeStruct((1, 1), jnp.float32),
        jax.ShapeDtypeStruct((1, 1), jnp.float32),
    ],
    scratch_shapes=[pltpu.VMEM((K,), jnp.float32)],
)


_NC, _NS = 2, 16
_NW = _NC * _NS
_BPW = T // _NW

@functools.cache
def _gather_rows_call():
    mesh = plsc.VectorSubcoreMesh(
        core_axis_name="c", subcore_axis_name="s", num_cores=_NC)

    @functools.partial(
        pl.kernel,
        mesh=mesh,
        compiler_params=pltpu.CompilerParams(use_tc_tiling_on_sc=False),
        out_type=jax.ShapeDtypeStruct((T, E), jnp.float32),
        scratch_types=[
            pltpu.VMEM((_BPW,), jnp.int32),
            pltpu.VMEM((_BPW, E), jnp.float32),
            pltpu.SemaphoreType.DMA,
        ],
    )
    def _gather_rows(emb_hbm, idx_hbm, out_hbm, idx_v, rows_v, sem):
        wid = lax.axis_index("s") * _NC + lax.axis_index("c")
        base = wid * _BPW
        pltpu.sync_copy(idx_hbm.at[pl.ds(base, _BPW)], idx_v)
        pltpu.async_copy(emb_hbm.at[idx_v], rows_v, sem).wait()
        pltpu.sync_copy(rows_v, out_hbm.at[pl.ds(base, _BPW)])

    return _gather_rows


def kernel(z, embedding):
    idx, loss, ppl = _vq_call(z, embedding)
    z_q = _gather_rows_call()(embedding, idx)
    return (z_q, loss[0, 0], idx, ppl[0, 0])


# speedup vs baseline: 1.1788x; 1.1695x over previous
import functools

import jax
import jax.numpy as jnp
from jax import lax
from jax.experimental import pallas as pl
from jax.experimental.pallas import tpu as pltpu
from jax.experimental.pallas import tpu_sc as plsc

K = 8192
E = 32
T = 4096
BT = 256
NT = T // BT
CC = 0.25


def _vq_body(z_ref, e_ref, idx_ref, loss_ref, ppl_ref, counts_ref):
    i = pl.program_id(0)
    z = z_ref[...]
    e = e_ref[...]
    mm = lax.dot_general(z * -2.0, e, (((1,), (1,)), ((), ())),
                         preferred_element_type=jnp.float32)
    z2 = jnp.sum(z * z, axis=1, keepdims=True)
    e2 = jnp.sum(e * e, axis=1)
    scores = z2 + e2[None, :] + mm
    minv = jnp.min(scores, axis=1)
    mask = scores == minv[:, None]
    kiota = lax.broadcasted_iota(jnp.int32, (BT, K), 1)
    idx = jnp.min(jnp.where(mask, kiota, K), axis=1)
    idx_ref[pl.ds(i * BT, BT)] = idx
    maskf = jnp.where(mask, 1.0, 0.0)
    cnt = lax.dot_general(jnp.ones((1, BT), jnp.float32), maskf,
                          (((1,), (0,)), ((), ())),
                          preferred_element_type=jnp.float32)[0]

    @pl.when(i == 0)
    def _():
        counts_ref[...] = cnt
        loss_ref[...] = jnp.sum(minv)[None, None]

    @pl.when(i > 0)
    def _():
        counts_ref[...] = counts_ref[...] + cnt
        loss_ref[...] = loss_ref[...] + jnp.sum(minv)[None, None]

    @pl.when(i == NT - 1)
    def _():
        p = counts_ref[...] * (1.0 / T)
        ppl_ref[...] = jnp.exp(-jnp.sum(p * jnp.log(p + 1e-10)))[None, None]
        loss_ref[...] = loss_ref[...] * (CC / (T * E))


_vq_call = pl.pallas_call(
    _vq_body,
    grid=(NT,),
    in_specs=[
        pl.BlockSpec((BT, E), lambda i: (i, 0)),
        pl.BlockSpec((K, E), lambda i: (0, 0)),
    ],
    out_specs=[
        pl.BlockSpec((T,), lambda i: (0,)),
        pl.BlockSpec((1, 1), lambda i: (0, 0)),
        pl.BlockSpec((1, 1), lambda i: (0, 0)),
    ],
    out_shape=[
        jax.ShapeDtypeStruct((T,), jnp.int32),
        jax.ShapeDtypeStruct((1, 1), jnp.float32),
        jax.ShapeDtypeStruct((1, 1), jnp.float32),
    ],
    scratch_shapes=[pltpu.VMEM((K,), jnp.float32)],
)


_NC, _NS = 2, 16
_NW = _NC * _NS
_BPW = T // _NW

@functools.cache
def _gather_rows_call():
    mesh = plsc.VectorSubcoreMesh(
        core_axis_name="c", subcore_axis_name="s", num_cores=_NC)

    @functools.partial(
        pl.kernel,
        mesh=mesh,
        compiler_params=pltpu.CompilerParams(use_tc_tiling_on_sc=False),
        out_type=jax.ShapeDtypeStruct((T, E), jnp.float32),
        scratch_types=[
            pltpu.VMEM((_BPW,), jnp.int32),
            pltpu.VMEM((_BPW, E), jnp.float32),
            pltpu.SemaphoreType.DMA,
        ],
    )
    def _gather_rows(emb_hbm, idx_hbm, out_hbm, idx_v, rows_v, sem):
        wid = lax.axis_index("s") * _NC + lax.axis_index("c")
        base = wid * _BPW
        pltpu.sync_copy(idx_hbm.at[pl.ds(base, _BPW)], idx_v)
        pltpu.async_copy(emb_hbm.at[idx_v], rows_v, sem).wait()
        pltpu.sync_copy(rows_v, out_hbm.at[pl.ds(base, _BPW)])

    return _gather_rows


def kernel(z, embedding):
    idx, loss, ppl = _vq_call(z, embedding)
    z_q = _gather_rows_call()(embedding, idx)
    return (z_q, loss[0, 0], idx, ppl[0, 0])


# speedup vs baseline: 1.2780x; 1.0842x over previous
import functools

import jax
import jax.numpy as jnp
from jax import lax
from jax.experimental import pallas as pl
from jax.experimental.pallas import tpu as pltpu
from jax.experimental.pallas import tpu_sc as plsc

K = 8192
E = 32
T = 4096
BT = 512
NT = T // BT
CC = 0.25


def _vq_body(z_ref, e_ref, idx_ref, loss_ref, ppl_ref, counts_ref):
    i = pl.program_id(0)
    z = z_ref[...]
    e = e_ref[...]
    mm = lax.dot_general(z * -2.0, e, (((1,), (1,)), ((), ())),
                         preferred_element_type=jnp.float32)
    z2 = jnp.sum(z * z, axis=1, keepdims=True)
    e2 = jnp.sum(e * e, axis=1)
    scores = z2 + e2[None, :] + mm
    minv = jnp.min(scores, axis=1)
    mask = scores == minv[:, None]
    kiota = lax.broadcasted_iota(jnp.int32, (BT, K), 1)
    idx = jnp.min(jnp.where(mask, kiota, K), axis=1)
    idx_ref[pl.ds(i * BT, BT)] = idx
    maskf = jnp.where(mask, 1.0, 0.0)
    cnt = lax.dot_general(jnp.ones((1, BT), jnp.float32), maskf,
                          (((1,), (0,)), ((), ())),
                          preferred_element_type=jnp.float32)[0]

    @pl.when(i == 0)
    def _():
        counts_ref[...] = cnt
        loss_ref[...] = jnp.sum(minv)[None, None]

    @pl.when(i > 0)
    def _():
        counts_ref[...] = counts_ref[...] + cnt
        loss_ref[...] = loss_ref[...] + jnp.sum(minv)[None, None]

    @pl.when(i == NT - 1)
    def _():
        p = counts_ref[...] * (1.0 / T)
        ppl_ref[...] = jnp.exp(-jnp.sum(p * jnp.log(p + 1e-10)))[None, None]
        loss_ref[...] = loss_ref[...] * (CC / (T * E))


_vq_call = pl.pallas_call(
    _vq_body,
    grid=(NT,),
    in_specs=[
        pl.BlockSpec((BT, E), lambda i: (i, 0)),
        pl.BlockSpec((K, E), lambda i: (0, 0)),
    ],
    out_specs=[
        pl.BlockSpec((T,), lambda i: (0,)),
        pl.BlockSpec((1, 1), lambda i: (0, 0)),
        pl.BlockSpec((1, 1), lambda i: (0, 0)),
    ],
    out_shape=[
        jax.ShapeDtypeStruct((T,), jnp.int32),
        jax.ShapeDtypeStruct((1, 1), jnp.float32),
        jax.ShapeDtypeStruct((1, 1), jnp.float32),
    ],
    scratch_shapes=[pltpu.VMEM((K,), jnp.float32)],
)


_NC, _NS = 2, 16
_NW = _NC * _NS
_BPW = T // _NW

@functools.cache
def _gather_rows_call():
    mesh = plsc.VectorSubcoreMesh(
        core_axis_name="c", subcore_axis_name="s", num_cores=_NC)

    @functools.partial(
        pl.kernel,
        mesh=mesh,
        compiler_params=pltpu.CompilerParams(use_tc_tiling_on_sc=False),
        out_type=jax.ShapeDtypeStruct((T, E), jnp.float32),
        scratch_types=[
            pltpu.VMEM((_BPW,), jnp.int32),
            pltpu.VMEM((_BPW, E), jnp.float32),
            pltpu.SemaphoreType.DMA,
        ],
    )
    def _gather_rows(emb_hbm, idx_hbm, out_hbm, idx_v, rows_v, sem):
        wid = lax.axis_index("s") * _NC + lax.axis_index("c")
        base = wid * _BPW
        pltpu.sync_copy(idx_hbm.at[pl.ds(base, _BPW)], idx_v)
        pltpu.async_copy(emb_hbm.at[idx_v], rows_v, sem).wait()
        pltpu.sync_copy(rows_v, out_hbm.at[pl.ds(base, _BPW)])

    return _gather_rows


def kernel(z, embedding):
    idx, loss, ppl = _vq_call(z, embedding)
    z_q = _gather_rows_call()(embedding, idx)
    return (z_q, loss[0, 0], idx, ppl[0, 0])
